# baseline (speedup 1.0000x reference)
import functools

import jax
import jax.numpy as jnp
from jax import lax
from jax.experimental import pallas as pl
from jax.experimental.pallas import tpu as pltpu
from jax.experimental.pallas import tpu_sc as plsc

_NUM_EXPERTS = 64
_TOP_K = 2
_ALPHA = 0.01


def _gate_body(hs_ref, w_ref, i1_ref, i2_ref, w1_ref, w2_ref, psum_ref):
    j = pl.program_id(1)
    ts = hs_ref.shape[1]
    hs = hs_ref[0]
    lt = jax.lax.dot_general(
        w_ref[...], hs, (((1,), (1,)), ((), ())),
        preferred_element_type=jnp.float32)

    m = jnp.max(lt, axis=0, keepdims=True)
    e = jnp.exp(lt - m)
    sig = jnp.sum(e, axis=0, keepdims=True)
    recip = 1.0 / sig
    s = e * recip

    iota = jax.lax.broadcasted_iota(jnp.int32, s.shape, 0)
    m1 = jnp.max(s, axis=0, keepdims=True)
    i1 = jnp.min(jnp.where(s == m1, iota, _NUM_EXPERTS), axis=0)
    masked = jnp.where(iota == i1[None, :], -1.0, s)
    m2 = jnp.max(masked, axis=0, keepdims=True)
    i2 = jnp.min(jnp.where(masked == m2, iota, _NUM_EXPERTS), axis=0)

    i1_ref[0] = i1[None]
    i2_ref[0] = i2[None]
    w1_ref[0] = m1
    w2_ref[0] = m2

    psum_part = jax.lax.dot_general(
        s, jnp.ones((ts, 1), jnp.float32), (((1,), (0,)), ((), ())),
        preferred_element_type=jnp.float32)

    @pl.when(j == 0)
    def _():
        psum_ref[...] = jnp.zeros_like(psum_ref)

    psum_ref[...] += psum_part[None]


def _tc_gate(hidden_states, weight):
    B, S, H = hidden_states.shape
    E = _NUM_EXPERTS
    TS = 1024
    nj = S // TS

    grid = (B, nj)
    out_shapes = (
        jax.ShapeDtypeStruct((B, 1, S), jnp.int32),
        jax.ShapeDtypeStruct((B, 1, S), jnp.int32),
        jax.ShapeDtypeStruct((B, 1, S), jnp.float32),
        jax.ShapeDtypeStruct((B, 1, S), jnp.float32),
        jax.ShapeDtypeStruct((B, E, 1), jnp.float32),
    )
    plane = pl.BlockSpec((1, 1, TS), lambda b, j: (b, 0, j))
    be = pl.BlockSpec((1, E, 1), lambda b, j: (b, 0, 0))
    return pl.pallas_call(
        _gate_body,
        grid=grid,
        in_specs=[
            pl.BlockSpec((1, TS, H), lambda b, j: (b, j, 0)),
            pl.BlockSpec((E, H), lambda b, j: (0, 0)),
        ],
        out_specs=(plane, plane, plane, plane, be),
        out_shape=out_shapes,
    )(hidden_states, weight)


def _make_sc_aux(chunk, per_batch):
    mesh = plsc.VectorSubcoreMesh(core_axis_name="c", subcore_axis_name="s")
    nsteps = chunk // 16

    @functools.partial(
        pl.kernel, mesh=mesh,
        out_type=jax.ShapeDtypeStruct((32, 16), jnp.float32),
        compiler_params=pltpu.CompilerParams(needs_layout_passes=False),
        scratch_types=[
            pltpu.VMEM((chunk,), jnp.int32),
            pltpu.VMEM((64,), jnp.float32),
            pltpu.VMEM((64,), jnp.float32),
            pltpu.VMEM((16,), jnp.float32),
        ],
    )
    def sc_aux(i1_hbm, i2_hbm, p_hbm, out_hbm, idx_v, p_v, hist_v, part_v):
        c = lax.axis_index("c")
        s = lax.axis_index("s")
        wid = s * 2 + c
        base = wid * chunk
        b = base // per_batch

        pltpu.sync_copy(p_hbm.at[pl.ds(b * 64, 64)], p_v)
        for k in range(4):
            hist_v[pl.ds(k * 16, 16)] = jnp.zeros((16,), jnp.float32)

        ones = jnp.full((16,), 1.0, jnp.float32)

        def accumulate(i, _):
            v = idx_v[pl.ds(i * 16, 16)]
            plsc.addupdate_scatter(hist_v, [v], ones)
            return 0

        pltpu.sync_copy(i1_hbm.at[pl.ds(base, chunk)], idx_v)
        lax.fori_loop(0, nsteps, accumulate, 0)
        pltpu.sync_copy(i2_hbm.at[pl.ds(base, chunk)], idx_v)
        lax.fori_loop(0, nsteps, accumulate, 0)

        acc = jnp.zeros((16,), jnp.float32)
        for k in range(4):
            acc = acc + hist_v[pl.ds(k * 16, 16)] * p_v[pl.ds(k * 16, 16)]
        part_v[...] = acc
        pltpu.sync_copy(part_v, out_hbm.at[wid])

    return sc_aux


def _make_sc_reduce(scale):
    mesh = plsc.VectorSubcoreMesh(core_axis_name="c", subcore_axis_name="s")

    @functools.partial(
        pl.kernel, mesh=mesh,
        out_type=jax.ShapeDtypeStruct((16,), jnp.float32),
        compiler_params=pltpu.CompilerParams(needs_layout_passes=False),
        scratch_types=[
            pltpu.VMEM((32, 16), jnp.float32),
            pltpu.VMEM((16,), jnp.float32),
        ],
    )
    def sc_reduce(parts_hbm, out_hbm, parts_v, out_v):
        c = lax.axis_index("c")
        s = lax.axis_index("s")

        @pl.when((c == 0) & (s == 0))
        def _():
            pltpu.sync_copy(parts_hbm, parts_v)
            acc = jnp.zeros((16,), jnp.float32)
            for k in range(32):
                acc = acc + parts_v[k]
            out_v[...] = jnp.zeros((16,), jnp.float32) + jnp.sum(acc) * scale
            pltpu.sync_copy(out_v, out_hbm)

    return sc_reduce


def kernel(hidden_states, weight):
    B, S, H = hidden_states.shape
    E = _NUM_EXPERTS
    scale = _ALPHA * E / (B * _TOP_K * float(S) ** 3)

    i1, i2, w1, w2, psum = _tc_gate(hidden_states, weight)

    sc_aux = _make_sc_aux((B * S) // 32, S)
    parts = sc_aux(i1.reshape(B * S), i2.reshape(B * S), psum.reshape(B * E))
    aux_vec = _make_sc_reduce(scale)(parts)
    aux_loss = aux_vec[0]

    topk_idx = jnp.stack([i1[:, 0, :], i2[:, 0, :]], axis=-1)
    topk_weight = jnp.stack([w1[:, 0, :], w2[:, 0, :]], axis=-1)
    return topk_idx, topk_weight, aux_loss


# speedup vs baseline: 1.1952x; 1.1952x over previous
import functools

import jax
import jax.numpy as jnp
from jax import lax
from jax.experimental import pallas as pl
from jax.experimental.pallas import tpu as pltpu
from jax.experimental.pallas import tpu_sc as plsc

_NUM_EXPERTS = 64
_TOP_K = 2
_ALPHA = 0.01


def _gate_body(hs_ref, w_ref, i1_ref, i2_ref, w1_ref, w2_ref, psum_ref):
    j = pl.program_id(1)
    ts = hs_ref.shape[1]
    hs = hs_ref[0]
    lt = jax.lax.dot_general(
        w_ref[...], hs, (((1,), (1,)), ((), ())),
        preferred_element_type=jnp.float32)

    m = jnp.max(lt, axis=0, keepdims=True)
    e = jnp.exp(lt - m)
    sig = jnp.sum(e, axis=0, keepdims=True)
    recip = 1.0 / sig
    s = e * recip

    iota = jax.lax.broadcasted_iota(jnp.int32, s.shape, 0)
    m1 = jnp.max(s, axis=0, keepdims=True)
    i1 = jnp.min(jnp.where(s == m1, iota, _NUM_EXPERTS), axis=0)
    masked = jnp.where(iota == i1[None, :], -1.0, s)
    m2 = jnp.max(masked, axis=0, keepdims=True)
    i2 = jnp.min(jnp.where(masked == m2, iota, _NUM_EXPERTS), axis=0)

    i1_ref[0] = i1[None]
    i2_ref[0] = i2[None]
    w1_ref[0] = m1
    w2_ref[0] = m2

    psum_part = jax.lax.dot_general(
        s, jnp.ones((ts, 1), jnp.float32), (((1,), (0,)), ((), ())),
        preferred_element_type=jnp.float32)

    @pl.when(j == 0)
    def _():
        psum_ref[...] = jnp.zeros_like(psum_ref)

    psum_ref[...] += psum_part[None]


def _tc_gate(hidden_states, weight):
    B, S, H = hidden_states.shape
    E = _NUM_EXPERTS
    TS = 2048
    nj = S // TS

    grid = (B, nj)
    out_shapes = (
        jax.ShapeDtypeStruct((B, 1, S), jnp.int32),
        jax.ShapeDtypeStruct((B, 1, S), jnp.int32),
        jax.ShapeDtypeStruct((B, 1, S), jnp.float32),
        jax.ShapeDtypeStruct((B, 1, S), jnp.float32),
        jax.ShapeDtypeStruct((B, E, 1), jnp.float32),
    )
    plane = pl.BlockSpec((1, 1, TS), lambda b, j: (b, 0, j))
    be = pl.BlockSpec((1, E, 1), lambda b, j: (b, 0, 0))
    return pl.pallas_call(
        _gate_body,
        grid=grid,
        in_specs=[
            pl.BlockSpec((1, TS, H), lambda b, j: (b, j, 0)),
            pl.BlockSpec((E, H), lambda b, j: (0, 0)),
        ],
        out_specs=(plane, plane, plane, plane, be),
        out_shape=out_shapes,
    )(hidden_states, weight)


def _make_sc_aux(chunk, per_batch):
    mesh = plsc.VectorSubcoreMesh(core_axis_name="c", subcore_axis_name="s")
    nsteps = chunk // 16

    @functools.partial(
        pl.kernel, mesh=mesh,
        out_type=jax.ShapeDtypeStruct((32, 16), jnp.float32),
        compiler_params=pltpu.CompilerParams(needs_layout_passes=False),
        scratch_types=[
            pltpu.VMEM((chunk,), jnp.int32),
            pltpu.VMEM((64,), jnp.float32),
            pltpu.VMEM((64,), jnp.float32),
            pltpu.VMEM((16,), jnp.float32),
        ],
    )
    def sc_aux(i1_hbm, i2_hbm, p_hbm, out_hbm, idx_v, p_v, hist_v, part_v):
        c = lax.axis_index("c")
        s = lax.axis_index("s")
        wid = s * 2 + c
        base = wid * chunk
        b = base // per_batch

        pltpu.sync_copy(p_hbm.at[pl.ds(b * 64, 64)], p_v)
        for k in range(4):
            hist_v[pl.ds(k * 16, 16)] = jnp.zeros((16,), jnp.float32)

        ones = jnp.full((16,), 1.0, jnp.float32)

        def accumulate(i, _):
            v = idx_v[pl.ds(i * 16, 16)]
            plsc.addupdate_scatter(hist_v, [v], ones)
            return 0

        pltpu.sync_copy(i1_hbm.at[pl.ds(base, chunk)], idx_v)
        lax.fori_loop(0, nsteps, accumulate, 0)
        pltpu.sync_copy(i2_hbm.at[pl.ds(base, chunk)], idx_v)
        lax.fori_loop(0, nsteps, accumulate, 0)

        acc = jnp.zeros((16,), jnp.float32)
        for k in range(4):
            acc = acc + hist_v[pl.ds(k * 16, 16)] * p_v[pl.ds(k * 16, 16)]
        part_v[...] = acc
        pltpu.sync_copy(part_v, out_hbm.at[wid])

    return sc_aux


def _reduce_body(scale, parts_ref, out_ref):
    out_ref[...] = jnp.sum(parts_ref[...]).reshape(1, 1) * scale


def _tc_reduce(parts, scale):
    return pl.pallas_call(
        functools.partial(_reduce_body, scale),
        out_shape=jax.ShapeDtypeStruct((1, 1), jnp.float32),
    )(parts)


def kernel(hidden_states, weight):
    B, S, H = hidden_states.shape
    E = _NUM_EXPERTS
    scale = _ALPHA * E / (B * _TOP_K * float(S) ** 3)

    i1, i2, w1, w2, psum = _tc_gate(hidden_states, weight)

    sc_aux = _make_sc_aux((B * S) // 32, S)
    parts = sc_aux(i1.reshape(B * S), i2.reshape(B * S), psum.reshape(B * E))
    aux_loss = _tc_reduce(parts, scale)[0, 0]

    topk_idx = jnp.stack([i1[:, 0, :], i2[:, 0, :]], axis=-1)
    topk_weight = jnp.stack([w1[:, 0, :], w2[:, 0, :]], axis=-1)
    return topk_idx, topk_weight, aux_loss


# speedup vs baseline: 1.2048x; 1.0081x over previous
import functools

import jax
import jax.numpy as jnp
from jax import lax
from jax.experimental import pallas as pl
from jax.experimental.pallas import tpu as pltpu
from jax.experimental.pallas import tpu_sc as plsc

_NUM_EXPERTS = 64
_TOP_K = 2
_ALPHA = 0.01


def _gate_body(hs_ref, w_ref, i1_ref, i2_ref, w1_ref, w2_ref, psum_ref):
    j = pl.program_id(1)
    ts = hs_ref.shape[1]
    hs = hs_ref[0]
    lt = jax.lax.dot_general(
        w_ref[...], hs, (((1,), (1,)), ((), ())),
        preferred_element_type=jnp.float32)

    m = jnp.max(lt, axis=0, keepdims=True)
    e = jnp.exp(lt - m)
    sig = jnp.sum(e, axis=0, keepdims=True)
    recip = 1.0 / sig
    s = e * recip

    iota = jax.lax.broadcasted_iota(jnp.int32, s.shape, 0)
    m1 = jnp.max(s, axis=0, keepdims=True)
    i1 = jnp.min(jnp.where(s == m1, iota, _NUM_EXPERTS), axis=0)
    masked = jnp.where(iota == i1[None, :], -1.0, s)
    m2 = jnp.max(masked, axis=0, keepdims=True)
    i2 = jnp.min(jnp.where(masked == m2, iota, _NUM_EXPERTS), axis=0)

    i1_ref[0] = i1[None]
    i2_ref[0] = i2[None]
    w1_ref[0] = m1
    w2_ref[0] = m2

    psum_part = jax.lax.dot_general(
        s, jnp.ones((ts, 1), jnp.float32), (((1,), (0,)), ((), ())),
        preferred_element_type=jnp.float32)

    @pl.when(j == 0)
    def _():
        psum_ref[...] = jnp.zeros_like(psum_ref)

    psum_ref[...] += psum_part[None]


def _tc_gate(hidden_states, weight):
    B, S, H = hidden_states.shape
    E = _NUM_EXPERTS
    TS = 4096
    nj = S // TS

    grid = (B, nj)
    out_shapes = (
        jax.ShapeDtypeStruct((B, 1, S), jnp.int32),
        jax.ShapeDtypeStruct((B, 1, S), jnp.int32),
        jax.ShapeDtypeStruct((B, 1, S), jnp.float32),
        jax.ShapeDtypeStruct((B, 1, S), jnp.float32),
        jax.ShapeDtypeStruct((B, E, 1), jnp.float32),
    )
    plane = pl.BlockSpec((1, 1, TS), lambda b, j: (b, 0, j))
    be = pl.BlockSpec((1, E, 1), lambda b, j: (b, 0, 0))
    return pl.pallas_call(
        _gate_body,
        grid=grid,
        in_specs=[
            pl.BlockSpec((1, TS, H), lambda b, j: (b, j, 0)),
            pl.BlockSpec((E, H), lambda b, j: (0, 0)),
        ],
        out_specs=(plane, plane, plane, plane, be),
        out_shape=out_shapes,
    )(hidden_states, weight)


def _make_sc_aux(chunk, per_batch):
    mesh = plsc.VectorSubcoreMesh(core_axis_name="c", subcore_axis_name="s")
    nsteps = chunk // 16

    @functools.partial(
        pl.kernel, mesh=mesh,
        out_type=jax.ShapeDtypeStruct((32, 16), jnp.float32),
        compiler_params=pltpu.CompilerParams(needs_layout_passes=False),
        scratch_types=[
            pltpu.VMEM((chunk,), jnp.int32),
            pltpu.VMEM((64,), jnp.float32),
            pltpu.VMEM((64,), jnp.float32),
            pltpu.VMEM((16,), jnp.float32),
        ],
    )
    def sc_aux(i1_hbm, i2_hbm, p_hbm, out_hbm, idx_v, p_v, hist_v, part_v):
        c = lax.axis_index("c")
        s = lax.axis_index("s")
        wid = s * 2 + c
        base = wid * chunk
        b = base // per_batch

        pltpu.sync_copy(p_hbm.at[pl.ds(b * 64, 64)], p_v)
        for k in range(4):
            hist_v[pl.ds(k * 16, 16)] = jnp.zeros((16,), jnp.float32)

        ones = jnp.full((16,), 1.0, jnp.float32)

        def accumulate(i, _):
            v = idx_v[pl.ds(i * 16, 16)]
            plsc.addupdate_scatter(hist_v, [v], ones)
            return 0

        pltpu.sync_copy(i1_hbm.at[pl.ds(base, chunk)], idx_v)
        lax.fori_loop(0, nsteps, accumulate, 0)
        pltpu.sync_copy(i2_hbm.at[pl.ds(base, chunk)], idx_v)
        lax.fori_loop(0, nsteps, accumulate, 0)

        acc = jnp.zeros((16,), jnp.float32)
        for k in range(4):
            acc = acc + hist_v[pl.ds(k * 16, 16)] * p_v[pl.ds(k * 16, 16)]
        part_v[...] = acc
        pltpu.sync_copy(part_v, out_hbm.at[wid])

    return sc_aux


def _reduce_body(scale, parts_ref, out_ref):
    out_ref[...] = jnp.sum(parts_ref[...]).reshape(1, 1) * scale


def _tc_reduce(parts, scale):
    return pl.pallas_call(
        functools.partial(_reduce_body, scale),
        out_shape=jax.ShapeDtypeStruct((1, 1), jnp.float32),
    )(parts)


def kernel(hidden_states, weight):
    B, S, H = hidden_states.shape
    E = _NUM_EXPERTS
    scale = _ALPHA * E / (B * _TOP_K * float(S) ** 3)

    i1, i2, w1, w2, psum = _tc_gate(hidden_states, weight)

    sc_aux = _make_sc_aux((B * S) // 32, S)
    parts = sc_aux(i1.reshape(B * S), i2.reshape(B * S), psum.reshape(B * E))
    aux_loss = _tc_reduce(parts, scale)[0, 0]

    topk_idx = jnp.stack([i1[:, 0, :], i2[:, 0, :]], axis=-1)
    topk_weight = jnp.stack([w1[:, 0, :], w2[:, 0, :]], axis=-1)
    return topk_idx, topk_weight, aux_loss
